# baseline (speedup 1.0000x reference)
import functools

import jax
import jax.numpy as jnp
from jax import lax
from jax.experimental import pallas as pl
from jax.experimental.pallas import tpu as pltpu
from jax.experimental.pallas import tpu_sc as plsc

LAT = 128
NLAYER = 3
LANES = 16
_C = 80
_NCH = 126
_LEAD = 4


def _sc_layer_fn(N, E):
    info = plsc.get_sparse_core_info()
    NC, NS = info.num_cores, info.num_subcores
    NW = NC * NS
    assert E == NW * _NCH * _C
    nzb = N // _C
    mesh = plsc.VectorSubcoreMesh(core_axis_name="c", subcore_axis_name="s")

    @functools.partial(
        pl.kernel,
        out_type=jax.ShapeDtypeStruct((NC, N, LAT), jnp.float32),
        mesh=mesh,
        scratch_types=[
            pltpu.VMEM((6, 2, _C), jnp.int32),
            pltpu.VMEM((6, _C), jnp.float32),
            pltpu.VMEM((3, _C), jnp.int32),
            pltpu.VMEM((_C, LAT), jnp.float32),
            pltpu.VMEM((_C, LAT), jnp.float32),
            pltpu.VMEM((_C, LAT), jnp.float32),
            pltpu.VMEM_SHARED((N, LAT), jnp.float32),
            pltpu.SemaphoreType.DMA((6,)),
            pltpu.SemaphoreType.DMA((3,)),
            pltpu.SemaphoreType.DMA((3,)),
        ],
    )
    def k(edges_h, w_h, x_h, out_h,
          ring, ringw, dcp, m0, m1, m2, acc, si, sg, ss):
        cid = lax.axis_index("c")
        sid = lax.axis_index("s")
        wid = sid * NC + cid
        M = (m0, m1, m2)

        def fire_edges(c, r):
            pltpu.async_copy(edges_h.at[wid, c], ring.at[r], si.at[r])
            pltpu.async_copy(w_h.at[wid, c], ringw.at[r], si.at[r])

        def edges_wait(c, r):
            pltpu.make_async_copy(
                edges_h.at[wid, c], ring.at[r], si.at[r]).wait()
            pltpu.make_async_copy(
                w_h.at[wid, c], ringw.at[r], si.at[r]).wait()

        def gather(c, r, q):
            pltpu.async_copy(x_h.at[ring.at[r, 0]], M[q], sg.at[q])

        def gather_wait(c, r, q):
            pltpu.make_async_copy(
                x_h.at[ring.at[r, 0]], M[q], sg.at[q]).wait()

        def scale_and_grab(r, q):
            def sgrp(g, _):
                lo = g * LANES
                wvec = ringw[r, pl.ds(lo, LANES)]
                dcp[q, pl.ds(lo, LANES)] = ring[r, 1, pl.ds(lo, LANES)]
                for t in range(LANES):
                    we = wvec[t]
                    e = lo + t
                    for j in range(LAT // LANES):
                        M[q][e, pl.ds(LANES * j, LANES)] = (
                            M[q][e, pl.ds(LANES * j, LANES)] * we)
                return _
            lax.fori_loop(0, _C // LANES, sgrp, None)

        def scatter(q):
            pltpu.sync_copy(M[q], acc.at[dcp.at[q]], add=True)

        def scatter_wait(q):
            pass

        for c0 in range(_LEAD):
            fire_edges(c0, c0)

        for m in M:
            def zfill(r_, _, m=m):
                for j in range(LAT // LANES):
                    m[r_, pl.ds(LANES * j, LANES)] = jnp.zeros(
                        (LANES,), jnp.float32)
                return _
            lax.fori_loop(0, _C, zfill, None)

        def dzfill(r_, _):
            for s in range(3):
                dcp[s, pl.ds(r_ * LANES, LANES)] = jnp.zeros(
                    (LANES,), jnp.int32)
            return _
        lax.fori_loop(0, _C // LANES, dzfill, None)

        def zcopy(t, _):
            blk = t * NS + sid
            @pl.when(blk < nzb)
            def _do():
                pltpu.sync_copy(m0, acc.at[pl.ds(blk * _C, _C)])
            return _
        lax.fori_loop(0, pl.cdiv(nzb, NS), zcopy, None)

        edges_wait(0, 0)
        gather(0, 0, 0)
        plsc.subcore_barrier()


        def body6(g, _):
            for dc in range(6):
                c = 6 * g + dc
                q = dc % 3
                qn = (dc + 1) % 3
                r = dc
                rn = (dc + 1) % 6
                r4 = (dc + _LEAD) % 6
                scatter_wait(qn)
                @pl.when(c + _LEAD < _NCH)
                def _fire():
                    fire_edges(c + _LEAD, r4)
                @pl.when(c + 1 < _NCH)
                def _pref():
                    edges_wait(c + 1, rn)
                    gather(c + 1, rn, qn)
                gather_wait(c, r, q)
                scale_and_grab(r, q)
                scatter(q)
            return _
        lax.fori_loop(0, _NCH // 6, body6, None)

        scatter_wait((_NCH - 2) % 3)
        scatter_wait((_NCH - 1) % 3)
        plsc.subcore_barrier()

        def wb_loop(t, _):
            blk = t * NS + sid
            @pl.when(blk < nzb)
            def _do():
                pltpu.sync_copy(acc.at[pl.ds(blk * _C, _C)],
                                out_h.at[cid, pl.ds(blk * _C, _C)])
            return _
        lax.fori_loop(0, pl.cdiv(nzb, NS), wb_loop, None)

    return k


def _merge(p, runsum):
    N, _ = runsum.shape
    blk = 400

    def mk(p_ref, rs_ref, x_ref, rs2_ref):
        a = p_ref[0] + p_ref[1]
        x_ref[...] = a
        rs2_ref[...] = rs_ref[...] + a

    return pl.pallas_call(
        mk,
        grid=(N // blk,),
        in_specs=[
            pl.BlockSpec((2, blk, LAT), lambda i: (0, i, 0)),
            pl.BlockSpec((blk, LAT), lambda i: (i, 0)),
        ],
        out_specs=[
            pl.BlockSpec((blk, LAT), lambda i: (i, 0)),
            pl.BlockSpec((blk, LAT), lambda i: (i, 0)),
        ],
        out_shape=[jax.ShapeDtypeStruct((N, LAT), jnp.float32)] * 2,
    )(p, runsum)


def kernel(edge_index, edge_weight, ini_embeds):
    N = ini_embeds.shape[0]
    E = edge_weight.shape[0]
    info = plsc.get_sparse_core_info()
    NW = info.num_cores * info.num_subcores
    ept = _NCH * _C
    pad = NW * ept - E
    src = jnp.concatenate([edge_index[0], jnp.zeros((pad,), edge_index.dtype)])
    dst = jnp.concatenate([edge_index[1], jnp.zeros((pad,), edge_index.dtype)])
    w = jnp.concatenate([edge_weight, jnp.zeros((pad,), edge_weight.dtype)])
    edges = jnp.stack(
        [src.reshape(NW, _NCH, _C), dst.reshape(NW, _NCH, _C)],
        axis=2)
    w = w.reshape(NW, _NCH, _C)
    layer = _sc_layer_fn(N, NW * ept)
    x = ini_embeds
    runsum = ini_embeds
    for _ in range(NLAYER):
        part = layer(edges, w, x)
        x, runsum = _merge(part, runsum)
    half = N // 2
    return runsum[:half], runsum[half:]


# speedup vs baseline: 1.7293x; 1.7293x over previous
import functools

import jax
import jax.numpy as jnp
from jax import lax
from jax.experimental import pallas as pl
from jax.experimental.pallas import tpu as pltpu
from jax.experimental.pallas import tpu_sc as plsc

LAT = 128
NLAYER = 3
LANES = 16
_C = 80
_BCH = 25
_NB = 5
_ZR = 40


def _sc_layer_fn(N, E):
    info = plsc.get_sparse_core_info()
    NC, NS = info.num_cores, info.num_subcores
    NW = NC * NS
    ept = _NB * _BCH * _C
    assert E == NW * ept
    nzb = N // _ZR
    mesh = plsc.VectorSubcoreMesh(core_axis_name="c", subcore_axis_name="s")

    @functools.partial(
        pl.kernel,
        out_type=jax.ShapeDtypeStruct((NC, N, LAT), jnp.float32),
        mesh=mesh,
        scratch_types=[
            pltpu.VMEM((_BCH * _C,), jnp.int32),
            pltpu.VMEM((_BCH * _C,), jnp.int32),
            pltpu.VMEM((2, _BCH, _C), jnp.int32),
            pltpu.VMEM((_BCH * _C,), jnp.float32),
            pltpu.VMEM((_BCH * _C,), jnp.float32),
            pltpu.VMEM((3, _C), jnp.int32),
            pltpu.VMEM((_C, LAT), jnp.float32),
            pltpu.VMEM((_C, LAT), jnp.float32),
            pltpu.VMEM((_C, LAT), jnp.float32),
            pltpu.VMEM_SHARED((N, LAT), jnp.float32),
            pltpu.SemaphoreType.DMA((2,)),
            pltpu.SemaphoreType.DMA((3,)),
            pltpu.SemaphoreType.DMA((3,)),
        ],
    )
    def k(src_h, dst_h, w_h, x_h, out_h,
          srcb0, srcb1, dstb, wb0, wb1, dcp, m0, m1, m2, acc, si, sg, ss):
        cid = lax.axis_index("c")
        sid = lax.axis_index("s")
        wid = sid * NC + cid
        M = (m0, m1, m2)
        SRCB = (srcb0, srcb1)
        WB = (wb0, wb1)

        def fire_block(b, par):
            return [
                pltpu.async_copy(src_h.at[wid * _NB + b], SRCB[par], si.at[par]),
                pltpu.async_copy(dst_h.at[wid, b], dstb.at[par], si.at[par]),
                pltpu.async_copy(w_h.at[wid * _NB + b], WB[par], si.at[par]),
            ]

        def gather(par, c, q):
            pltpu.async_copy(
                x_h.at[SRCB[par].at[pl.ds(c * _C, _C)]], M[q], sg.at[q])

        def gather_wait(par, c, q):
            pltpu.make_async_copy(
                x_h.at[SRCB[par].at[pl.ds(c * _C, _C)]], M[q], sg.at[q]).wait()

        def scale_and_grab(par, c, q):
            def sgrp(g, _):
                lo = g * LANES
                wvec = WB[par][pl.ds(c * _C + lo, LANES)]
                dcp[q, pl.ds(lo, LANES)] = dstb[par, c, pl.ds(lo, LANES)]
                for t in range(LANES):
                    we = wvec[t]
                    e = lo + t
                    for j in range(LAT // LANES):
                        M[q][e, pl.ds(LANES * j, LANES)] = (
                            M[q][e, pl.ds(LANES * j, LANES)] * we)
                return _
            lax.fori_loop(0, _C // LANES, sgrp, None)

        def scatter(q):
            pltpu.async_copy(M[q], acc.at[dcp.at[q]], ss.at[q], add=True)

        def scatter_wait(q):
            pltpu.make_async_copy(M[q], acc.at[dcp.at[q]], ss.at[q]).wait()

        h0 = fire_block(0, 0)

        def dzfill(r_, _):
            for s in range(3):
                dcp[s, pl.ds(r_ * LANES, LANES)] = jnp.zeros(
                    (LANES,), jnp.int32)
            return _
        lax.fori_loop(0, _C // LANES, dzfill, None)

        def zfill(r_, _):
            for j in range(LAT // LANES):
                m0[r_, pl.ds(LANES * j, LANES)] = jnp.zeros(
                    (LANES,), jnp.float32)
            return _
        lax.fori_loop(0, _C, zfill, None)

        def zcopy(t, _):
            blk = t * NS + sid
            @pl.when(blk < nzb)
            def _do():
                pltpu.sync_copy(m0.at[pl.ds(0, _ZR)],
                                acc.at[pl.ds(blk * _ZR, _ZR)])
            return _
        lax.fori_loop(0, pl.cdiv(nzb, NS), zcopy, None)

        for h in h0:
            h.wait()
        plsc.subcore_barrier()
        pltpu.async_copy(m0, acc.at[dcp.at[0]], ss.at[1], add=True)
        pltpu.async_copy(m0, acc.at[dcp.at[0]], ss.at[2], add=True)
        gather(0, 0, 0)

        for b in range(_NB):
            p = b % 2
            np_ = 1 - p
            hnext = fire_block(b + 1, np_) if b + 1 < _NB else None

            def body3(g, _, b=b, p=p):
                for dc in range(3):
                    c = 3 * g + dc
                    q = (b + dc) % 3
                    qn = (q + 1) % 3
                    scatter_wait(qn)
                    gather(p, c + 1, qn)
                    gather_wait(p, c, q)
                    scale_and_grab(p, c, q)
                    scatter(q)
                return _
            lax.fori_loop(0, (_BCH - 1) // 3, body3, None)

            q = b % 3
            qn = (q + 1) % 3
            if hnext is not None:
                for h in hnext:
                    h.wait()
                scatter_wait(qn)
                gather(np_, 0, qn)
            else:
                scatter_wait(qn)
            gather_wait(p, _BCH - 1, q)
            scale_and_grab(p, _BCH - 1, q)
            scatter(q)

        scatter_wait((_NB * _BCH - 2) % 3)
        scatter_wait((_NB * _BCH - 1) % 3)
        plsc.subcore_barrier()

        def wb_loop(t, _):
            blk = t * NS + sid
            @pl.when(blk < nzb)
            def _do():
                pltpu.sync_copy(acc.at[pl.ds(blk * _ZR, _ZR)],
                                out_h.at[cid, pl.ds(blk * _ZR, _ZR)])
            return _
        lax.fori_loop(0, pl.cdiv(nzb, NS), wb_loop, None)

    return k


def _merge(p, runsum):
    N, _ = runsum.shape
    blk = 400

    def mk(p_ref, rs_ref, x_ref, rs2_ref):
        a = p_ref[0] + p_ref[1]
        x_ref[...] = a
        rs2_ref[...] = rs_ref[...] + a

    return pl.pallas_call(
        mk,
        grid=(N // blk,),
        in_specs=[
            pl.BlockSpec((2, blk, LAT), lambda i: (0, i, 0)),
            pl.BlockSpec((blk, LAT), lambda i: (i, 0)),
        ],
        out_specs=[
            pl.BlockSpec((blk, LAT), lambda i: (i, 0)),
            pl.BlockSpec((blk, LAT), lambda i: (i, 0)),
        ],
        out_shape=[jax.ShapeDtypeStruct((N, LAT), jnp.float32)] * 2,
    )(p, runsum)


def kernel(edge_index, edge_weight, ini_embeds):
    N = ini_embeds.shape[0]
    E = edge_weight.shape[0]
    info = plsc.get_sparse_core_info()
    NW = info.num_cores * info.num_subcores
    ept = _NB * _BCH * _C
    pad = NW * ept - E
    src = jnp.concatenate([edge_index[0], jnp.zeros((pad,), edge_index.dtype)])
    dst = jnp.concatenate([edge_index[1], jnp.zeros((pad,), edge_index.dtype)])
    w = jnp.concatenate([edge_weight, jnp.zeros((pad,), edge_weight.dtype)])
    src = src.reshape(NW * _NB, _BCH * _C)
    dst = dst.reshape(NW, _NB, _BCH, _C)
    w = w.reshape(NW * _NB, _BCH * _C)
    layer = _sc_layer_fn(N, NW * ept)
    x = ini_embeds
    runsum = ini_embeds
    for _ in range(NLAYER):
        part = layer(src, dst, w, x)
        x, runsum = _merge(part, runsum)
    half = N // 2
    return runsum[:half], runsum[half:]
